# baseline (speedup 1.0000x reference)
import functools

import jax
import jax.numpy as jnp
from jax import lax
from jax.experimental import pallas as pl
from jax.experimental.pallas import tpu as pltpu
from jax.experimental.pallas import tpu_sc as plsc

L = 16
NC = 2
NS = 16
NW = NC * NS
BS = 16
NQ = 900
NT = 1600
NCHUNK = NT // L
QB = 4

SC_BATCHES = 2
TC_BATCHES = BS - SC_BATCHES
S_SLICES = NW // SC_BATCHES
Q_BASE = (NQ // S_SLICES) // QB * QB
Q_LAST = NQ - Q_BASE * (S_SLICES - 1)
OUT_TGT_W = -(-Q_LAST // 16) * 16

NQPAD = 1024
TC_QTILE = 128
NQB = NQPAD // TC_QTILE

POINT_W = 5.0
GIOU_W = 2.0

_TCX, _TCY, _TX0, _TY0, _TX1, _TY1, _TAREA = (k * NT for k in range(7))


def _vsqrt(d2):
    ih = plsc.bitcast(d2, jnp.int32)
    r = plsc.bitcast(jnp.int32(0x5F3759DF) - (ih >> 1), jnp.float32)
    hx = 0.5 * d2
    r = r * (1.5 - hx * r * r)
    r = r * (1.5 - hx * r * r)
    r = r * (1.5 - hx * r * r)
    s = d2 * r
    return 0.5 * (s + d2 / s)


_SC_KWARGS = dict(
    out_type=(
        jax.ShapeDtypeStruct((NW, NT), jnp.float32),
        jax.ShapeDtypeStruct((NW, NT), jnp.int32),
        jax.ShapeDtypeStruct((NW, OUT_TGT_W), jnp.int32),
    ),
    mesh=plsc.VectorSubcoreMesh(core_axis_name="c", subcore_axis_name="s",
                                num_cores=NC, num_subcores=NS),
    scratch_types=(
        pltpu.VMEM((NQ * 4,), jnp.float32),
        pltpu.VMEM((NT * 4,), jnp.float32),
        pltpu.VMEM((7 * NT,), jnp.float32),
        pltpu.VMEM((NT,), jnp.float32),
        pltpu.VMEM((NT,), jnp.int32),
        pltpu.VMEM((OUT_TGT_W,), jnp.int32),
    ),
    compiler_params=pltpu.CompilerParams(needs_layout_passes=False),
)


def _sc_matcher(pred_hbm, tgt_hbm, out_pval, out_pq, out_tgt,
                pred_v, tgt_v, pre_v, bval_v, bq_v, srcidx_v):
    w = lax.axis_index("c") * NS + lax.axis_index("s")
    b = w // S_SLICES
    sl = w % S_SLICES
    q0 = sl * Q_BASE
    count = Q_BASE + (Q_LAST - Q_BASE) * (sl == S_SLICES - 1).astype(jnp.int32)
    ngroups = count // QB
    iota = lax.iota(jnp.int32, L)
    inf_v = jnp.full((L,), jnp.inf, jnp.float32)

    pltpu.sync_copy(tgt_hbm, tgt_v)
    pltpu.sync_copy(pred_hbm.at[pl.ds(b * (NQ * 4), NQ * 4)], pred_v)

    def pre_body(cc, carry):
        base4 = (cc * L + iota) * 4
        tcx = plsc.load_gather(tgt_v, [base4])
        tcy = plsc.load_gather(tgt_v, [base4 + 1])
        tw = plsc.load_gather(tgt_v, [base4 + 2])
        th = plsc.load_gather(tgt_v, [base4 + 3])
        x0 = tcx - 0.5 * tw
        y0 = tcy - 0.5 * th
        x1 = tcx + 0.5 * tw
        y1 = tcy + 0.5 * th
        off = cc * L
        pre_v[pl.ds(off + _TCX, L)] = tcx
        pre_v[pl.ds(off + _TCY, L)] = tcy
        pre_v[pl.ds(off + _TX0, L)] = x0
        pre_v[pl.ds(off + _TY0, L)] = y0
        pre_v[pl.ds(off + _TX1, L)] = x1
        pre_v[pl.ds(off + _TY1, L)] = y1
        pre_v[pl.ds(off + _TAREA, L)] = (x1 - x0) * (y1 - y0)
        bval_v[pl.ds(off, L)] = inf_v
        bq_v[pl.ds(off, L)] = jnp.zeros((L,), jnp.int32)
        return carry

    lax.fori_loop(0, NCHUNK, pre_body, 0)

    def q_body(qg, acc):
        qi = qg * QB
        qabs = q0 + qi
        qbc = []
        for k in range(QB):
            qidx = jnp.full((L,), (qabs + k) * 4, jnp.int32)
            qcx = plsc.load_gather(pred_v, [qidx])
            qcy = plsc.load_gather(pred_v, [qidx + 1])
            qw = plsc.load_gather(pred_v, [qidx + 2])
            qh_ = plsc.load_gather(pred_v, [qidx + 3])
            qx0 = qcx - 0.5 * qw
            qy0 = qcy - 0.5 * qh_
            qx1 = qcx + 0.5 * qw
            qy1 = qcy + 0.5 * qh_
            qarea = (qx1 - qx0) * (qy1 - qy0)
            qbc.append((qcx, qcy, qx0, qy0, qx1, qy1, qarea))
        qvec = jnp.full((L,), qabs, jnp.int32)

        def one_chunk(cc, bv, bj):
            off = cc * L
            tcx = pre_v[pl.ds(off + _TCX, L)]
            tcy = pre_v[pl.ds(off + _TCY, L)]
            tx0 = pre_v[pl.ds(off + _TX0, L)]
            ty0 = pre_v[pl.ds(off + _TY0, L)]
            tx1 = pre_v[pl.ds(off + _TX1, L)]
            ty1 = pre_v[pl.ds(off + _TY1, L)]
            tarea = pre_v[pl.ds(off + _TAREA, L)]
            jvec = iota + off
            tv = bval_v[pl.ds(off, L)]
            tq = bq_v[pl.ds(off, L)]
            for k in range(QB):
                qcx, qcy, qx0, qy0, qx1, qy1, qarea = qbc[k]
                dx = qcx - tcx
                dy = qcy - tcy
                dist = _vsqrt(dx * dx + dy * dy + 1e-12)
                iw = jnp.maximum(jnp.minimum(qx1, tx1) - jnp.maximum(qx0, tx0), 0.0)
                ih = jnp.maximum(jnp.minimum(qy1, ty1) - jnp.maximum(qy0, ty0), 0.0)
                inter = iw * ih
                union = qarea + tarea - inter + 1e-16
                iou = inter / union
                ew = jnp.maximum(qx1, tx1) - jnp.minimum(qx0, tx0)
                eh = jnp.maximum(qy1, ty1) - jnp.minimum(qy0, ty0)
                earea = ew * eh + 1e-16
                giou = iou - (earea - union) / earea
                cost = POINT_W * dist - GIOU_W * giou
                m = cost < bv[k]
                bv[k] = jnp.where(m, cost, bv[k])
                bj[k] = jnp.where(m, jvec, bj[k])
                mt = cost < tv
                tv = jnp.where(mt, cost, tv)
                tq = jnp.where(mt, qvec + k, tq)
            bval_v[pl.ds(off, L)] = tv
            bq_v[pl.ds(off, L)] = tq
            return bv, bj

        def t_body(cc, carry):
            bv, bj = one_chunk(cc, list(carry[:QB]), list(carry[QB:]))
            return tuple(bv) + tuple(bj)

        init = (inf_v,) * QB + (jnp.zeros((L,), jnp.int32),) * QB
        res = lax.fori_loop(0, NCHUNK, t_body, init)
        accn = acc
        for k in range(QB):
            bv, bj = res[k], res[QB + k]
            minv = jnp.min(bv)
            jbest = jnp.min(jnp.where(bv == minv, bj, jnp.int32(1 << 30)))
            accn = jnp.where(iota == ((qi + k) % L), jbest, accn)

        @pl.when(qi % L == L - QB)
        def _():
            srcidx_v[pl.ds(qi - (L - QB), L)] = accn

        return accn

    acc = lax.fori_loop(0, ngroups, q_body, jnp.zeros((L,), jnp.int32))
    srcidx_v[pl.ds((count - 1) // L * L, L)] = acc

    pltpu.sync_copy(srcidx_v, out_tgt.at[w])
    pltpu.sync_copy(bval_v, out_pval.at[w])
    pltpu.sync_copy(bq_v, out_pq.at[w])


_sc_matcher_call = pl.kernel(_sc_matcher, **_SC_KWARGS)


def _tc_matcher(pred_ref, tgtt_ref, out_src_ref, out_tgt_ref, accv, accq):
    qb = pl.program_id(1)
    qcx = pred_ref[0, :, 0:1]
    qcy = pred_ref[0, :, 1:2]
    qw = pred_ref[0, :, 2:3]
    qh = pred_ref[0, :, 3:4]
    qx0 = qcx - 0.5 * qw
    qy0 = qcy - 0.5 * qh
    qx1 = qcx + 0.5 * qw
    qy1 = qcy + 0.5 * qh
    qarea = (qx1 - qx0) * (qy1 - qy0)
    tcx = tgtt_ref[0:1, :]
    tcy = tgtt_ref[1:2, :]
    tw = tgtt_ref[2:3, :]
    th = tgtt_ref[3:4, :]
    tx0 = tcx - 0.5 * tw
    ty0 = tcy - 0.5 * th
    tx1 = tcx + 0.5 * tw
    ty1 = tcy + 0.5 * th
    tarea = (tx1 - tx0) * (ty1 - ty0)

    dx = qcx - tcx
    dy = qcy - tcy
    dist = jnp.sqrt(dx * dx + dy * dy + 1e-12)
    iw = jnp.maximum(jnp.minimum(qx1, tx1) - jnp.maximum(qx0, tx0), 0.0)
    ih = jnp.maximum(jnp.minimum(qy1, ty1) - jnp.maximum(qy0, ty0), 0.0)
    inter = iw * ih
    union = qarea + tarea - inter + 1e-16
    iou = inter / union
    ew = jnp.maximum(qx1, tx1) - jnp.minimum(qx0, tx0)
    eh = jnp.maximum(qy1, ty1) - jnp.minimum(qy0, ty0)
    earea = ew * eh + 1e-16
    giou = iou - (earea - union) / earea
    cost = POINT_W * dist - GIOU_W * giou

    big = jnp.int32(1 << 30)
    minv = jnp.min(cost, axis=1, keepdims=True)
    lidx = lax.broadcasted_iota(jnp.int32, (TC_QTILE, NT), 1)
    jbest = jnp.min(jnp.where(cost == minv, lidx, big), axis=1, keepdims=True)
    out_tgt_ref[0, :, :] = jbest

    qgidx = qb * TC_QTILE + lax.broadcasted_iota(jnp.int32, (TC_QTILE, NT), 0)
    costm = jnp.where(qgidx < NQ, cost, jnp.inf)
    bminv = jnp.min(costm, axis=0, keepdims=True)
    bargq = jnp.min(jnp.where(costm == bminv, qgidx, big), axis=0, keepdims=True)

    @pl.when(qb == 0)
    def _():
        accv[...] = bminv
        accq[...] = bargq

    @pl.when(qb > 0)
    def _():
        m = bminv < accv[...]
        accv[...] = jnp.where(m, bminv, accv[...])
        accq[...] = jnp.where(m, bargq, accq[...])

    @pl.when(qb == NQB - 1)
    def _():
        out_src_ref[0] = accq[...]


_tc_matcher_call = pl.pallas_call(
    _tc_matcher,
    grid=(TC_BATCHES, NQB),
    in_specs=[
        pl.BlockSpec((1, TC_QTILE, 4), lambda b, q: (b, q, 0)),
        pl.BlockSpec((4, NT), lambda b, q: (0, 0)),
    ],
    out_specs=[
        pl.BlockSpec((1, 1, NT), lambda b, q: (b, 0, 0)),
        pl.BlockSpec((1, TC_QTILE, 1), lambda b, q: (b, q, 0)),
    ],
    out_shape=[
        jax.ShapeDtypeStruct((TC_BATCHES, 1, NT), jnp.int32),
        jax.ShapeDtypeStruct((TC_BATCHES, NQPAD, 1), jnp.int32),
    ],
    scratch_shapes=[
        pltpu.VMEM((1, NT), jnp.float32),
        pltpu.VMEM((1, NT), jnp.int32),
    ],
)


def kernel(pred_boxes, tgt_boxes):
    pred_sc = pred_boxes[:SC_BATCHES]
    pred_tc = pred_boxes[SC_BATCHES:]
    tgt_flat = tgt_boxes.reshape(-1)

    out_pval, out_pq, out_tgt = _sc_matcher_call(pred_sc.reshape(-1), tgt_flat)

    pred_tc_pad = jnp.pad(pred_tc, ((0, 0), (0, NQPAD - NQ), (0, 0)))
    tgt_t = tgt_boxes.T
    src_tc, tgtidx_tc = _tc_matcher_call(pred_tc_pad, tgt_t)

    vals = out_pval.reshape(SC_BATCHES, S_SLICES, NT)
    qs = out_pq.reshape(SC_BATCHES, S_SLICES, NT)
    jsel = jnp.argmin(vals, axis=1)
    src_sc = jnp.take_along_axis(qs, jsel[:, None, :], axis=1)[:, 0, :]
    rows = out_tgt.reshape(SC_BATCHES, S_SLICES, OUT_TGT_W)
    parts = [rows[:, j, :Q_BASE] for j in range(S_SLICES - 1)]
    parts.append(rows[:, S_SLICES - 1, :Q_LAST])
    tgtidx_sc = jnp.concatenate(parts, axis=1)

    indices_src = jnp.concatenate([src_sc, src_tc[:, 0, :]], axis=0)
    indices_tgt = jnp.concatenate(
        [tgtidx_sc, tgtidx_tc[:, :NQ, 0]], axis=0)
    return indices_src, indices_tgt


# speedup vs baseline: 1.2099x; 1.2099x over previous
import functools

import jax
import jax.numpy as jnp
from jax import lax
from jax.experimental import pallas as pl
from jax.experimental.pallas import tpu as pltpu
from jax.experimental.pallas import tpu_sc as plsc

L = 16
NC = 2
NS = 16
NW = NC * NS
BS = 16
NQ = 900
NT = 1600
NCHUNK = NT // L
QB = 4

SC_BATCHES = 4
TC_BATCHES = BS - SC_BATCHES
S_SLICES = NW // SC_BATCHES
Q_BASE = (NQ // S_SLICES) // QB * QB
Q_LAST = NQ - Q_BASE * (S_SLICES - 1)
OUT_TGT_W = -(-Q_LAST // 16) * 16

NQPAD = 1024
TC_QTILE = 256
NQB = NQPAD // TC_QTILE

POINT_W = 5.0
GIOU_W = 2.0

_TCX, _TCY, _TX0, _TY0, _TX1, _TY1, _TAREA = (k * NT for k in range(7))


def _vsqrt(d2):
    ih = plsc.bitcast(d2, jnp.int32)
    r = plsc.bitcast(jnp.int32(0x5F3759DF) - (ih >> 1), jnp.float32)
    hx = 0.5 * d2
    r = r * (1.5 - hx * r * r)
    r = r * (1.5 - hx * r * r)
    r = r * (1.5 - hx * r * r)
    s = d2 * r
    return 0.5 * (s + d2 / s)


_SC_KWARGS = dict(
    out_type=(
        jax.ShapeDtypeStruct((NW, NT), jnp.float32),
        jax.ShapeDtypeStruct((NW, NT), jnp.int32),
        jax.ShapeDtypeStruct((NW, OUT_TGT_W), jnp.int32),
    ),
    mesh=plsc.VectorSubcoreMesh(core_axis_name="c", subcore_axis_name="s",
                                num_cores=NC, num_subcores=NS),
    scratch_types=(
        pltpu.VMEM((NQ * 4,), jnp.float32),
        pltpu.VMEM((NT * 4,), jnp.float32),
        pltpu.VMEM((7 * NT,), jnp.float32),
        pltpu.VMEM((NT,), jnp.float32),
        pltpu.VMEM((NT,), jnp.int32),
        pltpu.VMEM((OUT_TGT_W,), jnp.int32),
    ),
    compiler_params=pltpu.CompilerParams(needs_layout_passes=False),
)


def _sc_matcher(pred_hbm, tgt_hbm, out_pval, out_pq, out_tgt,
                pred_v, tgt_v, pre_v, bval_v, bq_v, srcidx_v):
    w = lax.axis_index("c") * NS + lax.axis_index("s")
    b = w // S_SLICES
    sl = w % S_SLICES
    q0 = sl * Q_BASE
    count = Q_BASE + (Q_LAST - Q_BASE) * (sl == S_SLICES - 1).astype(jnp.int32)
    ngroups = count // QB
    iota = lax.iota(jnp.int32, L)
    inf_v = jnp.full((L,), jnp.inf, jnp.float32)

    pltpu.sync_copy(tgt_hbm, tgt_v)
    pltpu.sync_copy(pred_hbm.at[pl.ds(b * (NQ * 4), NQ * 4)], pred_v)

    def pre_body(cc, carry):
        base4 = (cc * L + iota) * 4
        tcx = plsc.load_gather(tgt_v, [base4])
        tcy = plsc.load_gather(tgt_v, [base4 + 1])
        tw = plsc.load_gather(tgt_v, [base4 + 2])
        th = plsc.load_gather(tgt_v, [base4 + 3])
        x0 = tcx - 0.5 * tw
        y0 = tcy - 0.5 * th
        x1 = tcx + 0.5 * tw
        y1 = tcy + 0.5 * th
        off = cc * L
        pre_v[pl.ds(off + _TCX, L)] = tcx
        pre_v[pl.ds(off + _TCY, L)] = tcy
        pre_v[pl.ds(off + _TX0, L)] = x0
        pre_v[pl.ds(off + _TY0, L)] = y0
        pre_v[pl.ds(off + _TX1, L)] = x1
        pre_v[pl.ds(off + _TY1, L)] = y1
        pre_v[pl.ds(off + _TAREA, L)] = (x1 - x0) * (y1 - y0)
        bval_v[pl.ds(off, L)] = inf_v
        bq_v[pl.ds(off, L)] = jnp.zeros((L,), jnp.int32)
        return carry

    lax.fori_loop(0, NCHUNK, pre_body, 0)

    def q_body(qg, acc):
        qi = qg * QB
        qabs = q0 + qi
        qbc = []
        for k in range(QB):
            qidx = jnp.full((L,), (qabs + k) * 4, jnp.int32)
            qcx = plsc.load_gather(pred_v, [qidx])
            qcy = plsc.load_gather(pred_v, [qidx + 1])
            qw = plsc.load_gather(pred_v, [qidx + 2])
            qh_ = plsc.load_gather(pred_v, [qidx + 3])
            qx0 = qcx - 0.5 * qw
            qy0 = qcy - 0.5 * qh_
            qx1 = qcx + 0.5 * qw
            qy1 = qcy + 0.5 * qh_
            qarea = (qx1 - qx0) * (qy1 - qy0)
            qbc.append((qcx, qcy, qx0, qy0, qx1, qy1, qarea))
        qvec = jnp.full((L,), qabs, jnp.int32)

        def one_chunk(cc, bv, bj):
            off = cc * L
            tcx = pre_v[pl.ds(off + _TCX, L)]
            tcy = pre_v[pl.ds(off + _TCY, L)]
            tx0 = pre_v[pl.ds(off + _TX0, L)]
            ty0 = pre_v[pl.ds(off + _TY0, L)]
            tx1 = pre_v[pl.ds(off + _TX1, L)]
            ty1 = pre_v[pl.ds(off + _TY1, L)]
            tarea = pre_v[pl.ds(off + _TAREA, L)]
            jvec = iota + off
            tv = bval_v[pl.ds(off, L)]
            tq = bq_v[pl.ds(off, L)]
            for k in range(QB):
                qcx, qcy, qx0, qy0, qx1, qy1, qarea = qbc[k]
                dx = qcx - tcx
                dy = qcy - tcy
                dist = _vsqrt(dx * dx + dy * dy + 1e-12)
                iw = jnp.maximum(jnp.minimum(qx1, tx1) - jnp.maximum(qx0, tx0), 0.0)
                ih = jnp.maximum(jnp.minimum(qy1, ty1) - jnp.maximum(qy0, ty0), 0.0)
                inter = iw * ih
                union = qarea + tarea - inter + 1e-16
                iou = inter / union
                ew = jnp.maximum(qx1, tx1) - jnp.minimum(qx0, tx0)
                eh = jnp.maximum(qy1, ty1) - jnp.minimum(qy0, ty0)
                earea = ew * eh + 1e-16
                giou = iou - (earea - union) / earea
                cost = POINT_W * dist - GIOU_W * giou
                m = cost < bv[k]
                bv[k] = jnp.where(m, cost, bv[k])
                bj[k] = jnp.where(m, jvec, bj[k])
                mt = cost < tv
                tv = jnp.where(mt, cost, tv)
                tq = jnp.where(mt, qvec + k, tq)
            bval_v[pl.ds(off, L)] = tv
            bq_v[pl.ds(off, L)] = tq
            return bv, bj

        def t_body(cc, carry):
            bv, bj = one_chunk(cc, list(carry[:QB]), list(carry[QB:]))
            return tuple(bv) + tuple(bj)

        init = (inf_v,) * QB + (jnp.zeros((L,), jnp.int32),) * QB
        res = lax.fori_loop(0, NCHUNK, t_body, init)
        accn = acc
        for k in range(QB):
            bv, bj = res[k], res[QB + k]
            minv = jnp.min(bv)
            jbest = jnp.min(jnp.where(bv == minv, bj, jnp.int32(1 << 30)))
            accn = jnp.where(iota == ((qi + k) % L), jbest, accn)

        @pl.when(qi % L == L - QB)
        def _():
            srcidx_v[pl.ds(qi - (L - QB), L)] = accn

        return accn

    acc = lax.fori_loop(0, ngroups, q_body, jnp.zeros((L,), jnp.int32))
    srcidx_v[pl.ds((count - 1) // L * L, L)] = acc

    pltpu.sync_copy(srcidx_v, out_tgt.at[w])
    pltpu.sync_copy(bval_v, out_pval.at[w])
    pltpu.sync_copy(bq_v, out_pq.at[w])


_sc_matcher_call = pl.kernel(_sc_matcher, **_SC_KWARGS)


def _tc_matcher(pred_ref, tgtt_ref, out_src_ref, out_tgt_ref, accv, accq):
    qb = pl.program_id(1)
    qcx = pred_ref[0, :, 0:1]
    qcy = pred_ref[0, :, 1:2]
    qw = pred_ref[0, :, 2:3]
    qh = pred_ref[0, :, 3:4]
    qx0 = qcx - 0.5 * qw
    qy0 = qcy - 0.5 * qh
    qx1 = qcx + 0.5 * qw
    qy1 = qcy + 0.5 * qh
    qarea = (qx1 - qx0) * (qy1 - qy0)
    tcx = tgtt_ref[0:1, :]
    tcy = tgtt_ref[1:2, :]
    tw = tgtt_ref[2:3, :]
    th = tgtt_ref[3:4, :]
    tx0 = tcx - 0.5 * tw
    ty0 = tcy - 0.5 * th
    tx1 = tcx + 0.5 * tw
    ty1 = tcy + 0.5 * th
    tarea = (tx1 - tx0) * (ty1 - ty0)

    dx = qcx - tcx
    dy = qcy - tcy
    dist = jnp.sqrt(dx * dx + dy * dy + 1e-12)
    iw = jnp.maximum(jnp.minimum(qx1, tx1) - jnp.maximum(qx0, tx0), 0.0)
    ih = jnp.maximum(jnp.minimum(qy1, ty1) - jnp.maximum(qy0, ty0), 0.0)
    inter = iw * ih
    union = qarea + tarea - inter + 1e-16
    iou = inter / union
    ew = jnp.maximum(qx1, tx1) - jnp.minimum(qx0, tx0)
    eh = jnp.maximum(qy1, ty1) - jnp.minimum(qy0, ty0)
    earea = ew * eh + 1e-16
    giou = iou - (earea - union) / earea
    cost = POINT_W * dist - GIOU_W * giou

    big = jnp.int32(1 << 30)
    minv = jnp.min(cost, axis=1, keepdims=True)
    lidx = lax.broadcasted_iota(jnp.int32, (TC_QTILE, NT), 1)
    jbest = jnp.min(jnp.where(cost == minv, lidx, big), axis=1, keepdims=True)
    out_tgt_ref[0, :, :] = jbest

    qgidx = qb * TC_QTILE + lax.broadcasted_iota(jnp.int32, (TC_QTILE, NT), 0)
    costm = jnp.where(qgidx < NQ, cost, jnp.inf)
    bminv = jnp.min(costm, axis=0, keepdims=True)
    bargq = jnp.min(jnp.where(costm == bminv, qgidx, big), axis=0, keepdims=True)

    @pl.when(qb == 0)
    def _():
        accv[...] = bminv
        accq[...] = bargq

    @pl.when(qb > 0)
    def _():
        m = bminv < accv[...]
        accv[...] = jnp.where(m, bminv, accv[...])
        accq[...] = jnp.where(m, bargq, accq[...])

    @pl.when(qb == NQB - 1)
    def _():
        out_src_ref[0] = accq[...]


_tc_matcher_call = pl.pallas_call(
    _tc_matcher,
    grid=(TC_BATCHES, NQB),
    in_specs=[
        pl.BlockSpec((1, TC_QTILE, 4), lambda b, q: (b, q, 0)),
        pl.BlockSpec((4, NT), lambda b, q: (0, 0)),
    ],
    out_specs=[
        pl.BlockSpec((1, 1, NT), lambda b, q: (b, 0, 0)),
        pl.BlockSpec((1, TC_QTILE, 1), lambda b, q: (b, q, 0)),
    ],
    out_shape=[
        jax.ShapeDtypeStruct((TC_BATCHES, 1, NT), jnp.int32),
        jax.ShapeDtypeStruct((TC_BATCHES, NQPAD, 1), jnp.int32),
    ],
    scratch_shapes=[
        pltpu.VMEM((1, NT), jnp.float32),
        pltpu.VMEM((1, NT), jnp.int32),
    ],
)


def kernel(pred_boxes, tgt_boxes):
    pred_sc = pred_boxes[:SC_BATCHES]
    pred_tc = pred_boxes[SC_BATCHES:]
    tgt_flat = tgt_boxes.reshape(-1)

    out_pval, out_pq, out_tgt = _sc_matcher_call(pred_sc.reshape(-1), tgt_flat)

    pred_tc_pad = jnp.pad(pred_tc, ((0, 0), (0, NQPAD - NQ), (0, 0)))
    tgt_t = tgt_boxes.T
    src_tc, tgtidx_tc = _tc_matcher_call(pred_tc_pad, tgt_t)

    vals = out_pval.reshape(SC_BATCHES, S_SLICES, NT)
    qs = out_pq.reshape(SC_BATCHES, S_SLICES, NT)
    jsel = jnp.argmin(vals, axis=1)
    src_sc = jnp.take_along_axis(qs, jsel[:, None, :], axis=1)[:, 0, :]
    rows = out_tgt.reshape(SC_BATCHES, S_SLICES, OUT_TGT_W)
    parts = [rows[:, j, :Q_BASE] for j in range(S_SLICES - 1)]
    parts.append(rows[:, S_SLICES - 1, :Q_LAST])
    tgtidx_sc = jnp.concatenate(parts, axis=1)

    indices_src = jnp.concatenate([src_sc, src_tc[:, 0, :]], axis=0)
    indices_tgt = jnp.concatenate(
        [tgtidx_sc, tgtidx_tc[:, :NQ, 0]], axis=0)
    return indices_src, indices_tgt


# speedup vs baseline: 1.2427x; 1.0271x over previous
import functools

import jax
import jax.numpy as jnp
from jax import lax
from jax.experimental import pallas as pl
from jax.experimental.pallas import tpu as pltpu
from jax.experimental.pallas import tpu_sc as plsc

L = 16
NC = 2
NS = 16
NW = NC * NS
BS = 16
NQ = 900
NT = 1600
NCHUNK = NT // L
QB = 4

SC_BATCHES = 4
TC_BATCHES = BS - SC_BATCHES
S_SLICES = NW // SC_BATCHES
Q_BASE = (NQ // S_SLICES) // QB * QB
Q_LAST = NQ - Q_BASE * (S_SLICES - 1)
OUT_TGT_W = -(-Q_LAST // 16) * 16

NQPAD = 1024
TC_QTILE = 512
NQB = NQPAD // TC_QTILE

POINT_W = 5.0
GIOU_W = 2.0

_TCX, _TCY, _TX0, _TY0, _TX1, _TY1, _TAREA = (k * NT for k in range(7))


def _vsqrt(d2):
    ih = plsc.bitcast(d2, jnp.int32)
    r = plsc.bitcast(jnp.int32(0x5F3759DF) - (ih >> 1), jnp.float32)
    hx = 0.5 * d2
    r = r * (1.5 - hx * r * r)
    r = r * (1.5 - hx * r * r)
    r = r * (1.5 - hx * r * r)
    s = d2 * r
    return 0.5 * (s + d2 / s)


_SC_KWARGS = dict(
    out_type=(
        jax.ShapeDtypeStruct((NW, NT), jnp.float32),
        jax.ShapeDtypeStruct((NW, NT), jnp.int32),
        jax.ShapeDtypeStruct((NW, OUT_TGT_W), jnp.int32),
    ),
    mesh=plsc.VectorSubcoreMesh(core_axis_name="c", subcore_axis_name="s",
                                num_cores=NC, num_subcores=NS),
    scratch_types=(
        pltpu.VMEM((NQ * 4,), jnp.float32),
        pltpu.VMEM((NT * 4,), jnp.float32),
        pltpu.VMEM((7 * NT,), jnp.float32),
        pltpu.VMEM((NT,), jnp.float32),
        pltpu.VMEM((NT,), jnp.int32),
        pltpu.VMEM((OUT_TGT_W,), jnp.int32),
    ),
    compiler_params=pltpu.CompilerParams(needs_layout_passes=False),
)


def _sc_matcher(pred_hbm, tgt_hbm, out_pval, out_pq, out_tgt,
                pred_v, tgt_v, pre_v, bval_v, bq_v, srcidx_v):
    w = lax.axis_index("c") * NS + lax.axis_index("s")
    b = w // S_SLICES
    sl = w % S_SLICES
    q0 = sl * Q_BASE
    count = Q_BASE + (Q_LAST - Q_BASE) * (sl == S_SLICES - 1).astype(jnp.int32)
    ngroups = count // QB
    iota = lax.iota(jnp.int32, L)
    inf_v = jnp.full((L,), jnp.inf, jnp.float32)

    pltpu.sync_copy(tgt_hbm, tgt_v)
    pltpu.sync_copy(pred_hbm.at[pl.ds(b * (NQ * 4), NQ * 4)], pred_v)

    def pre_body(cc, carry):
        base4 = (cc * L + iota) * 4
        tcx = plsc.load_gather(tgt_v, [base4])
        tcy = plsc.load_gather(tgt_v, [base4 + 1])
        tw = plsc.load_gather(tgt_v, [base4 + 2])
        th = plsc.load_gather(tgt_v, [base4 + 3])
        x0 = tcx - 0.5 * tw
        y0 = tcy - 0.5 * th
        x1 = tcx + 0.5 * tw
        y1 = tcy + 0.5 * th
        off = cc * L
        pre_v[pl.ds(off + _TCX, L)] = tcx
        pre_v[pl.ds(off + _TCY, L)] = tcy
        pre_v[pl.ds(off + _TX0, L)] = x0
        pre_v[pl.ds(off + _TY0, L)] = y0
        pre_v[pl.ds(off + _TX1, L)] = x1
        pre_v[pl.ds(off + _TY1, L)] = y1
        pre_v[pl.ds(off + _TAREA, L)] = (x1 - x0) * (y1 - y0)
        bval_v[pl.ds(off, L)] = inf_v
        bq_v[pl.ds(off, L)] = jnp.zeros((L,), jnp.int32)
        return carry

    lax.fori_loop(0, NCHUNK, pre_body, 0)

    def q_body(qg, acc):
        qi = qg * QB
        qabs = q0 + qi
        qbc = []
        for k in range(QB):
            qidx = jnp.full((L,), (qabs + k) * 4, jnp.int32)
            qcx = plsc.load_gather(pred_v, [qidx])
            qcy = plsc.load_gather(pred_v, [qidx + 1])
            qw = plsc.load_gather(pred_v, [qidx + 2])
            qh_ = plsc.load_gather(pred_v, [qidx + 3])
            qx0 = qcx - 0.5 * qw
            qy0 = qcy - 0.5 * qh_
            qx1 = qcx + 0.5 * qw
            qy1 = qcy + 0.5 * qh_
            qarea = (qx1 - qx0) * (qy1 - qy0)
            qbc.append((qcx, qcy, qx0, qy0, qx1, qy1, qarea))
        qvec = jnp.full((L,), qabs, jnp.int32)

        def one_chunk(cc, bv, bj):
            off = cc * L
            tcx = pre_v[pl.ds(off + _TCX, L)]
            tcy = pre_v[pl.ds(off + _TCY, L)]
            tx0 = pre_v[pl.ds(off + _TX0, L)]
            ty0 = pre_v[pl.ds(off + _TY0, L)]
            tx1 = pre_v[pl.ds(off + _TX1, L)]
            ty1 = pre_v[pl.ds(off + _TY1, L)]
            tarea = pre_v[pl.ds(off + _TAREA, L)]
            jvec = iota + off
            tv = bval_v[pl.ds(off, L)]
            tq = bq_v[pl.ds(off, L)]
            for k in range(QB):
                qcx, qcy, qx0, qy0, qx1, qy1, qarea = qbc[k]
                dx = qcx - tcx
                dy = qcy - tcy
                dist = _vsqrt(dx * dx + dy * dy + 1e-12)
                iw = jnp.maximum(jnp.minimum(qx1, tx1) - jnp.maximum(qx0, tx0), 0.0)
                ih = jnp.maximum(jnp.minimum(qy1, ty1) - jnp.maximum(qy0, ty0), 0.0)
                inter = iw * ih
                union = qarea + tarea - inter + 1e-16
                iou = inter / union
                ew = jnp.maximum(qx1, tx1) - jnp.minimum(qx0, tx0)
                eh = jnp.maximum(qy1, ty1) - jnp.minimum(qy0, ty0)
                earea = ew * eh + 1e-16
                giou = iou - (earea - union) / earea
                cost = POINT_W * dist - GIOU_W * giou
                m = cost < bv[k]
                bv[k] = jnp.where(m, cost, bv[k])
                bj[k] = jnp.where(m, jvec, bj[k])
                mt = cost < tv
                tv = jnp.where(mt, cost, tv)
                tq = jnp.where(mt, qvec + k, tq)
            bval_v[pl.ds(off, L)] = tv
            bq_v[pl.ds(off, L)] = tq
            return bv, bj

        def t_body(cc, carry):
            bv, bj = one_chunk(cc, list(carry[:QB]), list(carry[QB:]))
            return tuple(bv) + tuple(bj)

        init = (inf_v,) * QB + (jnp.zeros((L,), jnp.int32),) * QB
        res = lax.fori_loop(0, NCHUNK, t_body, init)
        accn = acc
        for k in range(QB):
            bv, bj = res[k], res[QB + k]
            minv = jnp.min(bv)
            jbest = jnp.min(jnp.where(bv == minv, bj, jnp.int32(1 << 30)))
            accn = jnp.where(iota == ((qi + k) % L), jbest, accn)

        @pl.when(qi % L == L - QB)
        def _():
            srcidx_v[pl.ds(qi - (L - QB), L)] = accn

        return accn

    acc = lax.fori_loop(0, ngroups, q_body, jnp.zeros((L,), jnp.int32))
    srcidx_v[pl.ds((count - 1) // L * L, L)] = acc

    pltpu.sync_copy(srcidx_v, out_tgt.at[w])
    pltpu.sync_copy(bval_v, out_pval.at[w])
    pltpu.sync_copy(bq_v, out_pq.at[w])


_sc_matcher_call = pl.kernel(_sc_matcher, **_SC_KWARGS)


def _tc_matcher(pred_ref, tgtt_ref, out_src_ref, out_tgt_ref, accv, accq):
    qb = pl.program_id(1)
    qcx = pred_ref[0, :, 0:1]
    qcy = pred_ref[0, :, 1:2]
    qw = pred_ref[0, :, 2:3]
    qh = pred_ref[0, :, 3:4]
    qx0 = qcx - 0.5 * qw
    qy0 = qcy - 0.5 * qh
    qx1 = qcx + 0.5 * qw
    qy1 = qcy + 0.5 * qh
    qarea = (qx1 - qx0) * (qy1 - qy0)
    tcx = tgtt_ref[0:1, :]
    tcy = tgtt_ref[1:2, :]
    tw = tgtt_ref[2:3, :]
    th = tgtt_ref[3:4, :]
    tx0 = tcx - 0.5 * tw
    ty0 = tcy - 0.5 * th
    tx1 = tcx + 0.5 * tw
    ty1 = tcy + 0.5 * th
    tarea = (tx1 - tx0) * (ty1 - ty0)

    dx = qcx - tcx
    dy = qcy - tcy
    dist = jnp.sqrt(dx * dx + dy * dy + 1e-12)
    iw = jnp.maximum(jnp.minimum(qx1, tx1) - jnp.maximum(qx0, tx0), 0.0)
    ih = jnp.maximum(jnp.minimum(qy1, ty1) - jnp.maximum(qy0, ty0), 0.0)
    inter = iw * ih
    union = qarea + tarea - inter + 1e-16
    iou = inter / union
    ew = jnp.maximum(qx1, tx1) - jnp.minimum(qx0, tx0)
    eh = jnp.maximum(qy1, ty1) - jnp.minimum(qy0, ty0)
    earea = ew * eh + 1e-16
    giou = iou - (earea - union) / earea
    cost = POINT_W * dist - GIOU_W * giou

    big = jnp.int32(1 << 30)
    minv = jnp.min(cost, axis=1, keepdims=True)
    lidx = lax.broadcasted_iota(jnp.int32, (TC_QTILE, NT), 1)
    jbest = jnp.min(jnp.where(cost == minv, lidx, big), axis=1, keepdims=True)
    out_tgt_ref[0, :, :] = jbest

    qgidx = qb * TC_QTILE + lax.broadcasted_iota(jnp.int32, (TC_QTILE, NT), 0)
    costm = jnp.where(qgidx < NQ, cost, jnp.inf)
    bminv = jnp.min(costm, axis=0, keepdims=True)
    bargq = jnp.min(jnp.where(costm == bminv, qgidx, big), axis=0, keepdims=True)

    @pl.when(qb == 0)
    def _():
        accv[...] = bminv
        accq[...] = bargq

    @pl.when(qb > 0)
    def _():
        m = bminv < accv[...]
        accv[...] = jnp.where(m, bminv, accv[...])
        accq[...] = jnp.where(m, bargq, accq[...])

    @pl.when(qb == NQB - 1)
    def _():
        out_src_ref[0] = accq[...]


_tc_matcher_call = pl.pallas_call(
    _tc_matcher,
    grid=(TC_BATCHES, NQB),
    in_specs=[
        pl.BlockSpec((1, TC_QTILE, 4), lambda b, q: (b, q, 0)),
        pl.BlockSpec((4, NT), lambda b, q: (0, 0)),
    ],
    out_specs=[
        pl.BlockSpec((1, 1, NT), lambda b, q: (b, 0, 0)),
        pl.BlockSpec((1, TC_QTILE, 1), lambda b, q: (b, q, 0)),
    ],
    out_shape=[
        jax.ShapeDtypeStruct((TC_BATCHES, 1, NT), jnp.int32),
        jax.ShapeDtypeStruct((TC_BATCHES, NQPAD, 1), jnp.int32),
    ],
    scratch_shapes=[
        pltpu.VMEM((1, NT), jnp.float32),
        pltpu.VMEM((1, NT), jnp.int32),
    ],
)


def kernel(pred_boxes, tgt_boxes):
    pred_sc = pred_boxes[:SC_BATCHES]
    pred_tc = pred_boxes[SC_BATCHES:]
    tgt_flat = tgt_boxes.reshape(-1)

    out_pval, out_pq, out_tgt = _sc_matcher_call(pred_sc.reshape(-1), tgt_flat)

    pred_tc_pad = jnp.pad(pred_tc, ((0, 0), (0, NQPAD - NQ), (0, 0)))
    tgt_t = tgt_boxes.T
    src_tc, tgtidx_tc = _tc_matcher_call(pred_tc_pad, tgt_t)

    vals = out_pval.reshape(SC_BATCHES, S_SLICES, NT)
    qs = out_pq.reshape(SC_BATCHES, S_SLICES, NT)
    jsel = jnp.argmin(vals, axis=1)
    src_sc = jnp.take_along_axis(qs, jsel[:, None, :], axis=1)[:, 0, :]
    rows = out_tgt.reshape(SC_BATCHES, S_SLICES, OUT_TGT_W)
    parts = [rows[:, j, :Q_BASE] for j in range(S_SLICES - 1)]
    parts.append(rows[:, S_SLICES - 1, :Q_LAST])
    tgtidx_sc = jnp.concatenate(parts, axis=1)

    indices_src = jnp.concatenate([src_sc, src_tc[:, 0, :]], axis=0)
    indices_tgt = jnp.concatenate(
        [tgtidx_sc, tgtidx_tc[:, :NQ, 0]], axis=0)
    return indices_src, indices_tgt
